# baseline (speedup 1.0000x reference)
import functools

import jax
import jax.numpy as jnp
from jax import lax
from jax.experimental import pallas as pl
from jax.experimental.pallas import tpu as pltpu
from jax.experimental.pallas import tpu_sc as plsc

NC = 2
NS = 16
NW = NC * NS
LANES = 16


def _sc_pool(ids2, scat2, table, B, H, E, SPG):
    SPW = B // NW
    CPW = SPW // SPG
    CH = E // LANES
    SPC = B // NC

    mesh = plsc.VectorSubcoreMesh(core_axis_name="c", subcore_axis_name="s")

    NBUF = 4

    @functools.partial(
        pl.kernel,
        mesh=mesh,
        out_type=jax.ShapeDtypeStruct((B, E), jnp.float32),
        scratch_types=(
            [pltpu.VMEM((CPW, SPG * H), jnp.int32),
             pltpu.VMEM((CPW, SPG * H), jnp.int32),
             pltpu.VMEM((SPW, E), jnp.float32),
             pltpu.VMEM_SHARED((SPC, E), jnp.float32)]
            + [pltpu.VMEM((SPG * H, E), jnp.float32) for _ in range(NBUF)]
            + [pltpu.SemaphoreType.DMA for _ in range(2 * NBUF)]
        ),
    )
    def k(table_hbm, ids_hbm, scat_hbm, out_hbm, idx_v, sidx_v, zbuf,
          shared, *rest):
        bufs = rest[:NBUF]
        gsems = rest[NBUF:2 * NBUF]
        ssems = rest[2 * NBUF:]
        sid = lax.axis_index("s")
        wid = sid * NC + lax.axis_index("c")
        base_chunk = wid * CPW
        pltpu.sync_copy(ids_hbm.at[pl.ds(base_chunk, CPW)], idx_v)
        pltpu.sync_copy(scat_hbm.at[pl.ds(base_chunk, CPW)], sidx_v)

        @pl.loop(0, SPW)
        def _(r):
            for c in range(CH):
                zbuf[r, pl.ds(c * LANES, LANES)] = jnp.zeros(
                    (LANES,), jnp.float32)

        pltpu.sync_copy(zbuf, shared.at[pl.ds(sid * SPW, SPW)])
        plsc.subcore_barrier()

        def start_gather(g, buf, sem):
            pltpu.async_copy(table_hbm.at[idx_v.at[g]], buf, sem)

        def wait_gather(g, buf, sem):
            pltpu.make_async_copy(table_hbm.at[idx_v.at[g]], buf, sem
                                  ).wait()

        def start_scatter(g, buf, sem):
            pltpu.async_copy(buf, shared.at[sidx_v.at[g]], sem, add=True)

        def wait_scatter(g, buf, sem):
            pltpu.make_async_copy(buf, shared.at[sidx_v.at[g]], sem).wait()

        for j in range(NBUF):
            start_gather(j, bufs[j], gsems[j])

        @pl.loop(0, CPW // NBUF - 1)
        def _(t):
            g0 = NBUF * t
            for j in range(NBUF):
                wait_gather(g0 + j, bufs[j], gsems[j])
                start_scatter(g0 + j, bufs[j], ssems[j])
            for j in range(NBUF):
                wait_scatter(g0 + j, bufs[j], ssems[j])
                start_gather(g0 + j + NBUF, bufs[j], gsems[j])

        g0 = CPW - NBUF
        for j in range(NBUF):
            wait_gather(g0 + j, bufs[j], gsems[j])
            start_scatter(g0 + j, bufs[j], ssems[j])
        for j in range(NBUF):
            wait_scatter(g0 + j, bufs[j], ssems[j])

        plsc.subcore_barrier()
        pltpu.sync_copy(shared.at[pl.ds(sid * SPW, SPW)],
                        out_hbm.at[pl.ds(wid * SPW, SPW)])

    return k(table, ids2, scat2)


def _mlp(pooled, W1, b1, W2, b2, B, H, E):
    HID = W1.shape[0]
    OUT = W2.shape[0]
    BB = 4096

    def body(x_ref, w1_ref, b1_ref, w2_ref, b2_ref, o_ref):
        x = x_ref[...] * (1.0 / H)
        h = lax.dot_general(x, w1_ref[...], (((1,), (1,)), ((), ())),
                            preferred_element_type=jnp.float32)
        h = jnp.maximum(h + b1_ref[...], 0.0)
        o = lax.dot_general(h, w2_ref[...], (((1,), (1,)), ((), ())),
                            preferred_element_type=jnp.float32)
        o_ref[...] = o + b2_ref[...]

    return pl.pallas_call(
        body,
        grid=(B // BB,),
        in_specs=[
            pl.BlockSpec((BB, E), lambda i: (i, 0)),
            pl.BlockSpec((HID, E), lambda i: (0, 0)),
            pl.BlockSpec((1, HID), lambda i: (0, 0)),
            pl.BlockSpec((OUT, HID), lambda i: (0, 0)),
            pl.BlockSpec((1, OUT), lambda i: (0, 0)),
        ],
        out_specs=pl.BlockSpec((BB, OUT), lambda i: (i, 0)),
        out_shape=jax.ShapeDtypeStruct((B, OUT), jnp.float32),
    )(pooled, W1, b1.reshape(1, HID), W2, b2.reshape(1, OUT))


def kernel(ids, emb_table, W1, b1, W2, b2):
    B, H = ids.shape
    E = emb_table.shape[1]
    SPG = 2
    SPW = B // NW
    CPW = SPW // SPG
    ids2 = ids.astype(jnp.int32).reshape(B // SPG, SPG * H)
    b2i = jnp.arange(B // SPG, dtype=jnp.int32)[:, None]
    kcol = jnp.arange(SPG * H, dtype=jnp.int32)[None, :]
    widv = b2i // CPW
    gv = b2i % CPW
    sidv = widv // NC
    scat2 = sidv * SPW + gv * SPG + kcol // H
    pooled = _sc_pool(ids2, scat2, emb_table, B, H, E, SPG)
    return _mlp(pooled, W1, b1, W2, b2, B, H, E)


# speedup vs baseline: 1.5166x; 1.5166x over previous
import functools

import jax
import jax.numpy as jnp
from jax import lax
from jax.experimental import pallas as pl
from jax.experimental.pallas import tpu as pltpu
from jax.experimental.pallas import tpu_sc as plsc

NC = 2
NS = 16
NW = NC * NS
LANES = 16


def _sc_pool(ids2, table, B, H, E, SPG):
    SPW = B // NW
    CPW = SPW // SPG
    CH = E // LANES

    mesh = plsc.VectorSubcoreMesh(core_axis_name="c", subcore_axis_name="s")

    NBUF = 4

    @functools.partial(
        pl.kernel,
        mesh=mesh,
        out_type=jax.ShapeDtypeStruct((B, E), jnp.float32),
        scratch_types=(
            [pltpu.VMEM((CPW, SPG * H), jnp.int32)]
            + [pltpu.VMEM((SPG * H, E), jnp.float32) for _ in range(NBUF)]
            + [pltpu.VMEM((SPW, E), jnp.float32)]
            + [pltpu.SemaphoreType.DMA for _ in range(NBUF)]
        ),
    )
    def k(table_hbm, ids_hbm, out_hbm, idx_v, *rest):
        bufs = rest[:NBUF]
        pool_v = rest[NBUF]
        sems = rest[NBUF + 1:]
        wid = lax.axis_index("s") * NC + lax.axis_index("c")
        base_chunk = wid * CPW
        pltpu.sync_copy(ids_hbm.at[pl.ds(base_chunk, CPW)], idx_v)

        def start(g, buf, sem):
            pltpu.async_copy(table_hbm.at[idx_v.at[g]], buf, sem)

        def wait(g, buf, sem):
            pltpu.make_async_copy(table_hbm.at[idx_v.at[g]], buf, sem
                                  ).wait()

        def accum(buf, g):
            for s in range(SPG):
                zero = jnp.zeros((LANES,), jnp.float32)

                def body(r, accs, s=s):
                    return tuple(
                        accs[c] + buf[r, pl.ds(c * LANES, LANES)]
                        for c in range(CH)
                    )
                accs = plsc.parallel_loop(
                    s * H, (s + 1) * H, unroll=10, carry=(zero,) * CH)(body)
                for c in range(CH):
                    pool_v[g * SPG + s, pl.ds(c * LANES, LANES)] = accs[c]

        for j in range(NBUF):
            start(j, bufs[j], sems[j])

        @pl.loop(0, CPW // NBUF - 1)
        def _(t):
            g0 = NBUF * t
            for j in range(NBUF):
                wait(g0 + j, bufs[j], sems[j])
                accum(bufs[j], g0 + j)
                start(g0 + j + NBUF, bufs[j], sems[j])

        for j in range(NBUF):
            g = CPW - NBUF + j
            wait(g, bufs[j], sems[j])
            accum(bufs[j], g)

        pltpu.sync_copy(pool_v, out_hbm.at[pl.ds(wid * SPW, SPW)])

    return k(table, ids2)


def _mlp(pooled, W1, b1, W2, b2, B, H, E):
    HID = W1.shape[0]
    OUT = W2.shape[0]
    BB = 4096

    def body(x_ref, w1_ref, b1_ref, w2_ref, b2_ref, o_ref):
        w1s = w1_ref[...] * (1.0 / H)
        ht = lax.dot_general(w1s, x_ref[...], (((1,), (1,)), ((), ())),
                             preferred_element_type=jnp.float32)
        ht = jnp.maximum(ht + b1_ref[...], 0.0)
        ot = lax.dot_general(w2_ref[...], ht, (((1,), (0,)), ((), ())),
                             preferred_element_type=jnp.float32)
        o_ref[...] = ot + b2_ref[...]

    out_t = pl.pallas_call(
        body,
        grid=(B // BB,),
        in_specs=[
            pl.BlockSpec((BB, E), lambda i: (i, 0)),
            pl.BlockSpec((HID, E), lambda i: (0, 0)),
            pl.BlockSpec((HID, 1), lambda i: (0, 0)),
            pl.BlockSpec((OUT, HID), lambda i: (0, 0)),
            pl.BlockSpec((OUT, 1), lambda i: (0, 0)),
        ],
        out_specs=pl.BlockSpec((OUT, BB), lambda i: (0, i)),
        out_shape=jax.ShapeDtypeStruct((OUT, B), jnp.float32),
    )(pooled, W1, b1.reshape(HID, 1), W2, b2.reshape(OUT, 1))
    return out_t.T


def kernel(ids, emb_table, W1, b1, W2, b2):
    B, H = ids.shape
    E = emb_table.shape[1]
    SPG = 2
    ids2 = ids.astype(jnp.int32).reshape(B // SPG, SPG * H)
    pooled = _sc_pool(ids2, emb_table, B, H, E, SPG)
    return _mlp(pooled, W1, b1, W2, b2, B, H, E)
